# Initial kernel scaffold; baseline (speedup 1.0000x reference)
import jax
import jax.numpy as jnp
from jax.experimental import pallas as pl


def kernel(x, edge_index, W1, a1_w, a1_b, W2, a2_w, a2_b):
    raise NotImplementedError("write your pallas kernel here")


# speedup vs baseline: 5.6686x; 5.6686x over previous
import functools

import jax
import jax.numpy as jnp
from jax import lax
from jax.experimental import pallas as pl
from jax.experimental.pallas import tpu as pltpu
from jax.experimental.pallas import tpu_sc as plsc

N = 10000
E = 320000
EP = E + N
C = 4096
EPAD = ((EP + C - 1) // C) * C
NP = 10016
NT = 32
FS = 4
NH4 = N * 4
ND4 = NP * 4
SLOPE = 0.2

_mesh = plsc.VectorSubcoreMesh(
    core_axis_name="c", subcore_axis_name="s", num_cores=2, num_subcores=16)


def _make_sc_gat(H):
    own_stride = NT // H

    @functools.partial(
        pl.kernel,
        mesh=_mesh,
        compiler_params=pltpu.CompilerParams(needs_layout_passes=False),
        out_type=(
            jax.ShapeDtypeStruct((NT, ND4), jnp.float32),
            jax.ShapeDtypeStruct((H, NP), jnp.float32),
        ),
        scratch_types=[
            pltpu.VMEM((NP,), jnp.float32),
            pltpu.VMEM((NP,), jnp.float32),
            pltpu.VMEM((NH4,), jnp.float32),
            pltpu.VMEM((ND4,), jnp.float32),
            pltpu.VMEM((NP,), jnp.float32),
            pltpu.VMEM((C,), jnp.int32),
            pltpu.VMEM((C,), jnp.int32),
            pltpu.VMEM((16,), jnp.float32),
        ],
    )
    def sc_gat(dst_hbm, src_hbm, sdst_hbm, ssrc_hbm, hsl_hbm, carr_hbm,
               out_hbm, den_hbm,
               sd_tab, ss_tab, h_tab, out_tab, den_tab, dst_buf, src_buf,
               c_buf):
        t = lax.axis_index("s") * 2 + lax.axis_index("c")
        head = t // own_stride
        own = (t % own_stride) == 0

        pltpu.sync_copy(sdst_hbm.at[head], sd_tab)
        pltpu.sync_copy(ssrc_hbm.at[head], ss_tab)
        pltpu.sync_copy(hsl_hbm.at[t], h_tab)
        pltpu.sync_copy(carr_hbm.at[head], c_buf)
        cv = c_buf[...]
        ownm = jnp.broadcast_to((t % own_stride).astype(jnp.int32), (16,)) == 0
        zf = jnp.zeros((16,), jnp.float32)

        def zero_out(i, carry):
            out_tab[pl.ds(i * 16, 16)] = zf
            return carry
        lax.fori_loop(0, ND4 // 16, zero_out, 0, unroll=4)

        def zero_den(i, carry):
            den_tab[pl.ds(i * 16, 16)] = zf
            return carry
        lax.fori_loop(0, NP // 16, zero_den, 0, unroll=4)

        def edge_block(j, carry):
            dv = dst_buf[pl.ds(j * 16, 16)]
            sv = src_buf[pl.ds(j * 16, 16)]
            sd = plsc.load_gather(sd_tab, [dv])
            ss = plsc.load_gather(ss_tab, [sv])
            z = sd + ss
            e = jnp.where(z >= 0, z, z * SLOPE) - cv
            p = jnp.exp(e)
            si = sv * 4
            di = dv * 4
            for f in range(FS):
                hf = plsc.load_gather(h_tab, [si + f])
                plsc.addupdate_scatter(out_tab, [di + f], p * hf)
            plsc.addupdate_scatter(den_tab, [dv], p, mask=ownm)
            return carry

        def chunk(b, carry):
            pltpu.sync_copy(dst_hbm.at[pl.ds(b * C, C)], dst_buf)
            pltpu.sync_copy(src_hbm.at[pl.ds(b * C, C)], src_buf)
            lax.fori_loop(0, C // 16, edge_block, 0, unroll=2)
            return carry
        lax.fori_loop(0, EPAD // C, chunk, 0)

        pltpu.sync_copy(out_tab, out_hbm.at[t])

        @pl.when(own)
        def _():
            pltpu.sync_copy(den_tab, den_hbm.at[head])

    return sc_gat


_sc_gat4 = _make_sc_gat(4)
_sc_gat1 = _make_sc_gat(1)


def _tc_in_body(x_ref, w_ref, m_ref, b_ref, h_ref, s_ref, mx_ref):
    h = lax.dot_general(x_ref[...], w_ref[...], (((1,), (1,)), ((), ())),
                        preferred_element_type=jnp.float32)
    h_ref[...] = h
    s = lax.dot_general(h, m_ref[...], (((1,), (0,)), ((), ())),
                        preferred_element_type=jnp.float32) + b_ref[...]
    s_ref[...] = s
    mx_ref[...] = jnp.max(s, axis=0, keepdims=True)


def _tc_in(x, W, M, b):
    k = M.shape[1]
    return pl.pallas_call(
        _tc_in_body,
        out_shape=(jax.ShapeDtypeStruct((N, 128), jnp.float32),
                   jax.ShapeDtypeStruct((N, k), jnp.float32),
                   jax.ShapeDtypeStruct((1, k), jnp.float32)),
    )(x, W, M, b)


def _tc_mid_body(o_ref, dex_ref, w_ref, m_ref, b_ref, h_ref, s_ref, mx_ref):
    xin = o_ref[...] / (dex_ref[...] + 1e-16)
    act = jnp.where(xin > 0, xin, jnp.exp(xin) - 1.0)
    h = lax.dot_general(act, w_ref[...], (((1,), (1,)), ((), ())),
                        preferred_element_type=jnp.float32)
    h_ref[...] = h
    s = lax.dot_general(h, m_ref[...], (((1,), (0,)), ((), ())),
                        preferred_element_type=jnp.float32) + b_ref[...]
    s_ref[...] = s
    mx_ref[...] = jnp.max(s, axis=0, keepdims=True)


def _tc_mid(o, dex, W, M, b):
    k = M.shape[1]
    return pl.pallas_call(
        _tc_mid_body,
        out_shape=(jax.ShapeDtypeStruct((N, 128), jnp.float32),
                   jax.ShapeDtypeStruct((N, k), jnp.float32),
                   jax.ShapeDtypeStruct((1, k), jnp.float32)),
    )(o, dex, W, M, b)


def _tc_out_body(o_ref, dex_ref, y_ref):
    v = o_ref[...] / (dex_ref[...] + 1e-16)
    m = jnp.max(v, axis=1, keepdims=True)
    lse = jnp.log(jnp.sum(jnp.exp(v - m), axis=1, keepdims=True)) + m
    y_ref[...] = v - lse


def _tc_out(o, dex):
    return pl.pallas_call(
        _tc_out_body,
        out_shape=jax.ShapeDtypeStruct((N, 128), jnp.float32),
    )(o, dex)


def _pad_table(sT):
    return jnp.pad(sT, ((0, 0), (0, NP - N)))


def _leaky(v):
    return jnp.where(v >= 0, v, v * SLOPE)


def kernel(x, edge_index, W1, a1_w, a1_b, W2, a2_w, a2_b):
    row = edge_index[0].astype(jnp.int32)
    col = edge_index[1].astype(jnp.int32)
    col = jnp.where(row != col, col, jnp.int32(N))
    loop = jnp.arange(N, dtype=jnp.int32)
    pad_n = EPAD - EP
    src = jnp.concatenate([row, loop, jnp.zeros((pad_n,), jnp.int32)])
    dst = jnp.concatenate([col, loop, jnp.full((pad_n,), N, jnp.int32)])

    H1, F1 = 4, 32
    eye1 = jnp.eye(H1, dtype=jnp.float32)
    M1 = jnp.concatenate([jnp.kron(eye1, a1_w[0, :F1][:, None]),
                          jnp.kron(eye1, a1_w[0, F1:][:, None])], axis=1)
    b1 = jnp.concatenate([jnp.broadcast_to(a1_b, (H1,)),
                          jnp.zeros((H1,), jnp.float32)])[None, :]
    h1, s1, mx1 = _tc_in(x, W1, M1, b1)

    sdst1 = _pad_table(s1[:, :H1].T)
    ssrc1 = _pad_table(s1[:, H1:].T)
    ce1 = _leaky(mx1[0, :H1] + mx1[0, H1:])
    carr1 = jnp.broadcast_to(ce1[:, None], (H1, 16))
    hsl1 = h1.reshape(N, NT, FS).transpose(1, 0, 2).reshape(NT, NH4)
    acc1, den1 = _sc_gat4(dst, src, sdst1, ssrc1, hsl1, carr1)
    o1 = acc1.reshape(NT, NP, FS)[:, :N, :].transpose(1, 0, 2).reshape(N, 128)
    dex1 = jnp.broadcast_to(den1[:, :N].T[:, :, None],
                            (N, H1, 128 // H1)).reshape(N, 128)

    H2, F2 = 1, 128
    M2 = jnp.stack([a2_w[0, :F2], a2_w[0, F2:]], axis=1)
    b2 = jnp.stack([a2_b[0], jnp.float32(0)])[None, :]
    h2, s2, mx2 = _tc_mid(o1, dex1, W2, M2, b2)

    sdst2 = _pad_table(s2[:, :1].T)
    ssrc2 = _pad_table(s2[:, 1:].T)
    ce2 = _leaky(mx2[0, :1] + mx2[0, 1:])
    carr2 = jnp.broadcast_to(ce2[:, None], (H2, 16))
    hsl2 = h2.reshape(N, NT, FS).transpose(1, 0, 2).reshape(NT, NH4)
    acc2, den2 = _sc_gat1(dst, src, sdst2, ssrc2, hsl2, carr2)
    o2 = acc2.reshape(NT, NP, FS)[:, :N, :].transpose(1, 0, 2).reshape(N, 128)
    dex2 = jnp.broadcast_to(den2[0, :N][:, None], (N, 128))

    return _tc_out(o2, dex2)


# speedup vs baseline: 5.6797x; 1.0020x over previous
import functools

import jax
import jax.numpy as jnp
from jax import lax
from jax.experimental import pallas as pl
from jax.experimental.pallas import tpu as pltpu
from jax.experimental.pallas import tpu_sc as plsc

N = 10000
E = 320000
EP = E + N
C = 4096
EPAD = ((EP + C - 1) // C) * C
NP = 10016
NT = 32
FS = 4
NH4 = N * 4
ND4 = NP * 4
SLOPE = 0.2

_mesh = plsc.VectorSubcoreMesh(
    core_axis_name="c", subcore_axis_name="s", num_cores=2, num_subcores=16)


def _make_sc_gat(H):
    own_stride = NT // H

    @functools.partial(
        pl.kernel,
        mesh=_mesh,
        compiler_params=pltpu.CompilerParams(needs_layout_passes=False),
        out_type=(
            jax.ShapeDtypeStruct((NT, ND4), jnp.float32),
            jax.ShapeDtypeStruct((H, NP), jnp.float32),
        ),
        scratch_types=[
            pltpu.VMEM((NP,), jnp.float32),
            pltpu.VMEM((NP,), jnp.float32),
            pltpu.VMEM((NH4,), jnp.float32),
            pltpu.VMEM((ND4,), jnp.float32),
            pltpu.VMEM((NP,), jnp.float32),
            pltpu.VMEM((C,), jnp.int32),
            pltpu.VMEM((C,), jnp.int32),
            pltpu.VMEM((16,), jnp.float32),
        ],
    )
    def sc_gat(dst_hbm, src_hbm, sdst_hbm, ssrc_hbm, hsl_hbm, carr_hbm,
               out_hbm, den_hbm,
               sd_tab, ss_tab, h_tab, out_tab, den_tab, dst_buf, src_buf,
               c_buf):
        t = lax.axis_index("s") * 2 + lax.axis_index("c")
        head = t // own_stride
        own = (t % own_stride) == 0

        pltpu.sync_copy(sdst_hbm.at[head], sd_tab)
        pltpu.sync_copy(ssrc_hbm.at[head], ss_tab)
        pltpu.sync_copy(hsl_hbm.at[t], h_tab)
        pltpu.sync_copy(carr_hbm.at[head], c_buf)
        cv = c_buf[...]
        ownm = jnp.broadcast_to((t % own_stride).astype(jnp.int32), (16,)) == 0
        zf = jnp.zeros((16,), jnp.float32)

        def zero_out(i, carry):
            out_tab[pl.ds(i * 16, 16)] = zf
            return carry
        lax.fori_loop(0, ND4 // 16, zero_out, 0, unroll=4)

        def zero_den(i, carry):
            den_tab[pl.ds(i * 16, 16)] = zf
            return carry
        lax.fori_loop(0, NP // 16, zero_den, 0, unroll=4)

        def edge_block(j, carry):
            dv = dst_buf[pl.ds(j * 16, 16)]
            sv = src_buf[pl.ds(j * 16, 16)]
            sd = plsc.load_gather(sd_tab, [dv])
            ss = plsc.load_gather(ss_tab, [sv])
            z = sd + ss
            e = jnp.where(z >= 0, z, z * SLOPE) - cv
            p = jnp.exp(e)
            si = sv * 4
            di = dv * 4
            for f in range(FS):
                hf = plsc.load_gather(h_tab, [si + f])
                plsc.addupdate_scatter(out_tab, [di + f], p * hf)
            plsc.addupdate_scatter(den_tab, [dv], p, mask=ownm)
            return carry

        def chunk(b, carry):
            pltpu.sync_copy(dst_hbm.at[pl.ds(b * C, C)], dst_buf)
            pltpu.sync_copy(src_hbm.at[pl.ds(b * C, C)], src_buf)
            lax.fori_loop(0, C // 16, edge_block, 0, unroll=8)
            return carry
        lax.fori_loop(0, EPAD // C, chunk, 0)

        pltpu.sync_copy(out_tab, out_hbm.at[t])

        @pl.when(own)
        def _():
            pltpu.sync_copy(den_tab, den_hbm.at[head])

    return sc_gat


_sc_gat4 = _make_sc_gat(4)
_sc_gat1 = _make_sc_gat(1)


def _tc_in_body(x_ref, w_ref, m_ref, b_ref, h_ref, s_ref, mx_ref):
    h = lax.dot_general(x_ref[...], w_ref[...], (((1,), (1,)), ((), ())),
                        preferred_element_type=jnp.float32)
    h_ref[...] = h
    s = lax.dot_general(h, m_ref[...], (((1,), (0,)), ((), ())),
                        preferred_element_type=jnp.float32) + b_ref[...]
    s_ref[...] = s
    mx_ref[...] = jnp.max(s, axis=0, keepdims=True)


def _tc_in(x, W, M, b):
    k = M.shape[1]
    return pl.pallas_call(
        _tc_in_body,
        out_shape=(jax.ShapeDtypeStruct((N, 128), jnp.float32),
                   jax.ShapeDtypeStruct((N, k), jnp.float32),
                   jax.ShapeDtypeStruct((1, k), jnp.float32)),
    )(x, W, M, b)


def _tc_mid_body(o_ref, dex_ref, w_ref, m_ref, b_ref, h_ref, s_ref, mx_ref):
    xin = o_ref[...] / (dex_ref[...] + 1e-16)
    act = jnp.where(xin > 0, xin, jnp.exp(xin) - 1.0)
    h = lax.dot_general(act, w_ref[...], (((1,), (1,)), ((), ())),
                        preferred_element_type=jnp.float32)
    h_ref[...] = h
    s = lax.dot_general(h, m_ref[...], (((1,), (0,)), ((), ())),
                        preferred_element_type=jnp.float32) + b_ref[...]
    s_ref[...] = s
    mx_ref[...] = jnp.max(s, axis=0, keepdims=True)


def _tc_mid(o, dex, W, M, b):
    k = M.shape[1]
    return pl.pallas_call(
        _tc_mid_body,
        out_shape=(jax.ShapeDtypeStruct((N, 128), jnp.float32),
                   jax.ShapeDtypeStruct((N, k), jnp.float32),
                   jax.ShapeDtypeStruct((1, k), jnp.float32)),
    )(o, dex, W, M, b)


def _tc_out_body(o_ref, dex_ref, y_ref):
    v = o_ref[...] / (dex_ref[...] + 1e-16)
    m = jnp.max(v, axis=1, keepdims=True)
    lse = jnp.log(jnp.sum(jnp.exp(v - m), axis=1, keepdims=True)) + m
    y_ref[...] = v - lse


def _tc_out(o, dex):
    return pl.pallas_call(
        _tc_out_body,
        out_shape=jax.ShapeDtypeStruct((N, 128), jnp.float32),
    )(o, dex)


def _pad_table(sT):
    return jnp.pad(sT, ((0, 0), (0, NP - N)))


def _leaky(v):
    return jnp.where(v >= 0, v, v * SLOPE)


def kernel(x, edge_index, W1, a1_w, a1_b, W2, a2_w, a2_b):
    row = edge_index[0].astype(jnp.int32)
    col = edge_index[1].astype(jnp.int32)
    col = jnp.where(row != col, col, jnp.int32(N))
    loop = jnp.arange(N, dtype=jnp.int32)
    pad_n = EPAD - EP
    src = jnp.concatenate([row, loop, jnp.zeros((pad_n,), jnp.int32)])
    dst = jnp.concatenate([col, loop, jnp.full((pad_n,), N, jnp.int32)])

    H1, F1 = 4, 32
    eye1 = jnp.eye(H1, dtype=jnp.float32)
    M1 = jnp.concatenate([jnp.kron(eye1, a1_w[0, :F1][:, None]),
                          jnp.kron(eye1, a1_w[0, F1:][:, None])], axis=1)
    b1 = jnp.concatenate([jnp.broadcast_to(a1_b, (H1,)),
                          jnp.zeros((H1,), jnp.float32)])[None, :]
    h1, s1, mx1 = _tc_in(x, W1, M1, b1)

    sdst1 = _pad_table(s1[:, :H1].T)
    ssrc1 = _pad_table(s1[:, H1:].T)
    ce1 = _leaky(mx1[0, :H1] + mx1[0, H1:])
    carr1 = jnp.broadcast_to(ce1[:, None], (H1, 16))
    hsl1 = h1.reshape(N, NT, FS).transpose(1, 0, 2).reshape(NT, NH4)
    acc1, den1 = _sc_gat4(dst, src, sdst1, ssrc1, hsl1, carr1)
    o1 = acc1.reshape(NT, NP, FS)[:, :N, :].transpose(1, 0, 2).reshape(N, 128)
    dex1 = jnp.broadcast_to(den1[:, :N].T[:, :, None],
                            (N, H1, 128 // H1)).reshape(N, 128)

    H2, F2 = 1, 128
    M2 = jnp.stack([a2_w[0, :F2], a2_w[0, F2:]], axis=1)
    b2 = jnp.stack([a2_b[0], jnp.float32(0)])[None, :]
    h2, s2, mx2 = _tc_mid(o1, dex1, W2, M2, b2)

    sdst2 = _pad_table(s2[:, :1].T)
    ssrc2 = _pad_table(s2[:, 1:].T)
    ce2 = _leaky(mx2[0, :1] + mx2[0, 1:])
    carr2 = jnp.broadcast_to(ce2[:, None], (H2, 16))
    hsl2 = h2.reshape(N, NT, FS).transpose(1, 0, 2).reshape(NT, NH4)
    acc2, den2 = _sc_gat1(dst, src, sdst2, ssrc2, hsl2, carr2)
    o2 = acc2.reshape(NT, NP, FS)[:, :N, :].transpose(1, 0, 2).reshape(N, 128)
    dex2 = jnp.broadcast_to(den2[0, :N][:, None], (N, 128))

    return _tc_out(o2, dex2)


# speedup vs baseline: 8.9410x; 1.5742x over previous
import functools

import jax
import jax.numpy as jnp
from jax import lax
from jax.experimental import pallas as pl
from jax.experimental.pallas import tpu as pltpu
from jax.experimental.pallas import tpu_sc as plsc

N = 10000
E = 320000
EP = E + N
C = 4096
EPAD = ((EP + C - 1) // C) * C
NP = 10016
NT = 32
FS = 4
NH4 = N * 4
ND4 = NP * 4
SLOPE = 0.2

_mesh = plsc.VectorSubcoreMesh(
    core_axis_name="c", subcore_axis_name="s", num_cores=2, num_subcores=16)


def _make_sc_gat(H):
    own_stride = NT // H

    @functools.partial(
        pl.kernel,
        mesh=_mesh,
        compiler_params=pltpu.CompilerParams(needs_layout_passes=False),
        out_type=(
            jax.ShapeDtypeStruct((NT, ND4), jnp.float32),
            jax.ShapeDtypeStruct((H, NP), jnp.float32),
        ),
        scratch_types=[
            pltpu.VMEM((NP,), jnp.float32),
            pltpu.VMEM((NP,), jnp.float32),
            pltpu.VMEM((NH4,), jnp.float32),
            pltpu.VMEM((ND4,), jnp.float32),
            pltpu.VMEM((NP,), jnp.float32),
            pltpu.VMEM((2 * C,), jnp.int32),
            pltpu.VMEM((16,), jnp.float32),
        ],
    )
    def sc_gat(edge_hbm, sdst_hbm, ssrc_hbm, hsl_hbm, carr_hbm,
               out_hbm, den_hbm,
               sd_tab, ss_tab, h_tab, out_tab, den_tab, e_buf, c_buf):
        t = lax.axis_index("s") * 2 + lax.axis_index("c")
        head = t // own_stride
        own = (t % own_stride) == 0

        pltpu.sync_copy(sdst_hbm.at[head], sd_tab)
        pltpu.sync_copy(ssrc_hbm.at[head], ss_tab)
        pltpu.sync_copy(hsl_hbm.at[t], h_tab)
        pltpu.sync_copy(carr_hbm.at[head], c_buf)
        cv = c_buf[...]
        ownm = jnp.broadcast_to((t % own_stride).astype(jnp.int32), (16,)) == 0
        zf = jnp.zeros((16,), jnp.float32)

        @plsc.parallel_loop(0, ND4 // 16, unroll=4)
        def zero_out(i):
            out_tab[pl.ds(i * 16, 16)] = zf

        @plsc.parallel_loop(0, NP // 16, unroll=4)
        def zero_den(i):
            den_tab[pl.ds(i * 16, 16)] = zf

        def chunk(b, carry):
            pltpu.sync_copy(edge_hbm.at[pl.ds(b * 2 * C, 2 * C)], e_buf)

            @plsc.parallel_loop(0, C // 16, unroll=8)
            def edge_block(j):
                dv = e_buf[pl.ds(j * 16, 16)]
                sv = e_buf[pl.ds(C + j * 16, 16)]
                sd = plsc.load_gather(sd_tab, [dv])
                ss = plsc.load_gather(ss_tab, [sv])
                z = sd + ss
                e = jnp.where(z >= 0, z, z * SLOPE) - cv
                p = jnp.exp(e)
                si = sv * 4
                di = dv * 4
                for f in range(FS):
                    hf = plsc.load_gather(h_tab, [si + f])
                    plsc.addupdate_scatter(out_tab, [di + f], p * hf)
                plsc.addupdate_scatter(den_tab, [dv], p, mask=ownm)

            return carry
        lax.fori_loop(0, EPAD // C, chunk, 0)

        pltpu.sync_copy(out_tab, out_hbm.at[t])

        @pl.when(own)
        def _():
            pltpu.sync_copy(den_tab, den_hbm.at[head])

    return sc_gat


_sc_gat4 = _make_sc_gat(4)
_sc_gat1 = _make_sc_gat(1)


def _tc_in_body(x_ref, w_ref, m_ref, b_ref, h_ref, s_ref, mx_ref):
    h = lax.dot_general(x_ref[...], w_ref[...], (((1,), (1,)), ((), ())),
                        preferred_element_type=jnp.float32)
    h_ref[...] = h
    s = lax.dot_general(h, m_ref[...], (((1,), (0,)), ((), ())),
                        preferred_element_type=jnp.float32) + b_ref[...]
    s_ref[...] = s
    mx_ref[...] = jnp.max(s, axis=0, keepdims=True)


def _tc_in(x, W, M, b):
    k = M.shape[1]
    return pl.pallas_call(
        _tc_in_body,
        out_shape=(jax.ShapeDtypeStruct((N, 128), jnp.float32),
                   jax.ShapeDtypeStruct((N, k), jnp.float32),
                   jax.ShapeDtypeStruct((1, k), jnp.float32)),
    )(x, W, M, b)


def _tc_mid_body(o_ref, dex_ref, w_ref, m_ref, b_ref, h_ref, s_ref, mx_ref):
    xin = o_ref[...] / (dex_ref[...] + 1e-16)
    act = jnp.where(xin > 0, xin, jnp.exp(xin) - 1.0)
    h = lax.dot_general(act, w_ref[...], (((1,), (1,)), ((), ())),
                        preferred_element_type=jnp.float32)
    h_ref[...] = h
    s = lax.dot_general(h, m_ref[...], (((1,), (0,)), ((), ())),
                        preferred_element_type=jnp.float32) + b_ref[...]
    s_ref[...] = s
    mx_ref[...] = jnp.max(s, axis=0, keepdims=True)


def _tc_mid(o, dex, W, M, b):
    k = M.shape[1]
    return pl.pallas_call(
        _tc_mid_body,
        out_shape=(jax.ShapeDtypeStruct((N, 128), jnp.float32),
                   jax.ShapeDtypeStruct((N, k), jnp.float32),
                   jax.ShapeDtypeStruct((1, k), jnp.float32)),
    )(o, dex, W, M, b)


def _tc_out_body(o_ref, dex_ref, y_ref):
    v = o_ref[...] / (dex_ref[...] + 1e-16)
    m = jnp.max(v, axis=1, keepdims=True)
    lse = jnp.log(jnp.sum(jnp.exp(v - m), axis=1, keepdims=True)) + m
    y_ref[...] = v - lse


def _tc_out(o, dex):
    return pl.pallas_call(
        _tc_out_body,
        out_shape=jax.ShapeDtypeStruct((N, 128), jnp.float32),
    )(o, dex)


def _pad_table(sT):
    return jnp.pad(sT, ((0, 0), (0, NP - N)))


def _leaky(v):
    return jnp.where(v >= 0, v, v * SLOPE)


def kernel(x, edge_index, W1, a1_w, a1_b, W2, a2_w, a2_b):
    row = edge_index[0].astype(jnp.int32)
    col = edge_index[1].astype(jnp.int32)
    col = jnp.where(row != col, col, jnp.int32(N))
    loop = jnp.arange(N, dtype=jnp.int32)
    pad_n = EPAD - EP
    src = jnp.concatenate([row, loop, jnp.zeros((pad_n,), jnp.int32)])
    dst = jnp.concatenate([col, loop, jnp.full((pad_n,), N, jnp.int32)])
    edges = jnp.stack([dst.reshape(-1, C), src.reshape(-1, C)],
                      axis=1).reshape(-1)

    H1, F1 = 4, 32
    eye1 = jnp.eye(H1, dtype=jnp.float32)
    M1 = jnp.concatenate([jnp.kron(eye1, a1_w[0, :F1][:, None]),
                          jnp.kron(eye1, a1_w[0, F1:][:, None])], axis=1)
    b1 = jnp.concatenate([jnp.broadcast_to(a1_b, (H1,)),
                          jnp.zeros((H1,), jnp.float32)])[None, :]
    h1, s1, mx1 = _tc_in(x, W1, M1, b1)

    sdst1 = _pad_table(s1[:, :H1].T)
    ssrc1 = _pad_table(s1[:, H1:].T)
    ce1 = _leaky(mx1[0, :H1] + mx1[0, H1:])
    carr1 = jnp.broadcast_to(ce1[:, None], (H1, 16))
    hsl1 = h1.reshape(N, NT, FS).transpose(1, 0, 2).reshape(NT, NH4)
    acc1, den1 = _sc_gat4(edges, sdst1, ssrc1, hsl1, carr1)
    o1 = acc1.reshape(NT, NP, FS)[:, :N, :].transpose(1, 0, 2).reshape(N, 128)
    dex1 = jnp.broadcast_to(den1[:, :N].T[:, :, None],
                            (N, H1, 128 // H1)).reshape(N, 128)

    H2, F2 = 1, 128
    M2 = jnp.stack([a2_w[0, :F2], a2_w[0, F2:]], axis=1)
    b2 = jnp.stack([a2_b[0], jnp.float32(0)])[None, :]
    h2, s2, mx2 = _tc_mid(o1, dex1, W2, M2, b2)

    sdst2 = _pad_table(s2[:, :1].T)
    ssrc2 = _pad_table(s2[:, 1:].T)
    ce2 = _leaky(mx2[0, :1] + mx2[0, 1:])
    carr2 = jnp.broadcast_to(ce2[:, None], (H2, 16))
    hsl2 = h2.reshape(N, NT, FS).transpose(1, 0, 2).reshape(NT, NH4)
    acc2, den2 = _sc_gat1(edges, sdst2, ssrc2, hsl2, carr2)
    o2 = acc2.reshape(NT, NP, FS)[:, :N, :].transpose(1, 0, 2).reshape(N, 128)
    dex2 = jnp.broadcast_to(den2[0, :N][:, None], (N, 128))

    return _tc_out(o2, dex2)


# speedup vs baseline: 18.5031x; 2.0695x over previous
import functools

import jax
import jax.numpy as jnp
from jax import lax
from jax.experimental import pallas as pl
from jax.experimental.pallas import tpu as pltpu
from jax.experimental.pallas import tpu_sc as plsc

N = 10000
E = 320000
EP = E + N
C = 4096
EPAD = ((EP + C - 1) // C) * C
NP = 10016
NT = 32
FS = 4
NH4 = N * 4
ND4 = NP * 4
SLOPE = 0.2

_mesh = plsc.VectorSubcoreMesh(
    core_axis_name="c", subcore_axis_name="s", num_cores=2, num_subcores=16)


def _make_sc_gat(H):
    own_stride = NT // H

    @functools.partial(
        pl.kernel,
        mesh=_mesh,
        compiler_params=pltpu.CompilerParams(needs_layout_passes=False),
        out_type=(
            jax.ShapeDtypeStruct((NT * FS, NP), jnp.float32),
            jax.ShapeDtypeStruct((H, NP), jnp.float32),
        ),
        scratch_types=[
            pltpu.VMEM((NP,), jnp.float32),
            pltpu.VMEM((NP,), jnp.float32),
            pltpu.VMEM((N,), jnp.float32),
            pltpu.VMEM((N,), jnp.float32),
            pltpu.VMEM((N,), jnp.float32),
            pltpu.VMEM((N,), jnp.float32),
            pltpu.VMEM((NP,), jnp.float32),
            pltpu.VMEM((NP,), jnp.float32),
            pltpu.VMEM((NP,), jnp.float32),
            pltpu.VMEM((NP,), jnp.float32),
            pltpu.VMEM((NP,), jnp.float32),
            pltpu.VMEM((2 * C,), jnp.int32),
            pltpu.VMEM((16,), jnp.float32),
        ],
    )
    def sc_gat(edge_hbm, sdst_hbm, ssrc_hbm, hsl_hbm, carr_hbm,
               out_hbm, den_hbm,
               sd_tab, ss_tab, h0, h1, h2, h3, o0, o1, o2, o3,
               den_tab, e_buf, c_buf):
        t = lax.axis_index("s") * 2 + lax.axis_index("c")
        head = t // own_stride
        own = (t % own_stride) == 0
        h_tabs = [h0, h1, h2, h3]
        o_tabs = [o0, o1, o2, o3]

        pltpu.sync_copy(sdst_hbm.at[head], sd_tab)
        pltpu.sync_copy(ssrc_hbm.at[head], ss_tab)
        for f in range(FS):
            pltpu.sync_copy(hsl_hbm.at[t * FS + f], h_tabs[f])
        pltpu.sync_copy(carr_hbm.at[head], c_buf)
        cv = c_buf[...]
        ownm = jnp.broadcast_to((t % own_stride).astype(jnp.int32), (16,)) == 0
        zf = jnp.zeros((16,), jnp.float32)

        for fz in range(FS):
            @plsc.parallel_loop(0, NP // 16, unroll=4)
            def zero_out(i, fz=fz):
                o_tabs[fz][pl.ds(i * 16, 16)] = zf

        @plsc.parallel_loop(0, NP // 16, unroll=4)
        def zero_den(i):
            den_tab[pl.ds(i * 16, 16)] = zf

        def chunk(b, carry):
            pltpu.sync_copy(edge_hbm.at[pl.ds(b * 2 * C, 2 * C)], e_buf)

            @plsc.parallel_loop(0, C // 16, unroll=8)
            def edge_block(j):
                dv = e_buf[pl.ds(j * 16, 16)]
                sv = e_buf[pl.ds(C + j * 16, 16)]
                sd = plsc.load_gather(sd_tab, [dv])
                ss = plsc.load_gather(ss_tab, [sv])
                z = sd + ss
                e = jnp.where(z >= 0, z, z * SLOPE) - cv
                p = jnp.exp(e)
                for f in range(FS):
                    hf = plsc.load_gather(h_tabs[f], [sv])
                    plsc.addupdate_scatter(o_tabs[f], [dv], p * hf)
                plsc.addupdate_scatter(den_tab, [dv], p, mask=ownm)

            return carry
        lax.fori_loop(0, EPAD // C, chunk, 0)

        for f in range(FS):
            pltpu.sync_copy(o_tabs[f], out_hbm.at[t * FS + f])

        @pl.when(own)
        def _():
            pltpu.sync_copy(den_tab, den_hbm.at[head])

    return sc_gat


_sc_gat4 = _make_sc_gat(4)
_sc_gat1 = _make_sc_gat(1)


def _tc_in_body(x_ref, w_ref, m_ref, b_ref, h_ref, s_ref, mx_ref):
    h = lax.dot_general(x_ref[...], w_ref[...], (((1,), (1,)), ((), ())),
                        preferred_element_type=jnp.float32)
    h_ref[...] = h
    s = lax.dot_general(h, m_ref[...], (((1,), (0,)), ((), ())),
                        preferred_element_type=jnp.float32) + b_ref[...]
    s_ref[...] = s
    mx_ref[...] = jnp.max(s, axis=0, keepdims=True)


def _tc_in(x, W, M, b):
    k = M.shape[1]
    return pl.pallas_call(
        _tc_in_body,
        out_shape=(jax.ShapeDtypeStruct((N, 128), jnp.float32),
                   jax.ShapeDtypeStruct((N, k), jnp.float32),
                   jax.ShapeDtypeStruct((1, k), jnp.float32)),
    )(x, W, M, b)


def _tc_mid_body(o_ref, dex_ref, w_ref, m_ref, b_ref, h_ref, s_ref, mx_ref):
    xin = o_ref[...] / (dex_ref[...] + 1e-16)
    act = jnp.where(xin > 0, xin, jnp.exp(xin) - 1.0)
    h = lax.dot_general(act, w_ref[...], (((1,), (1,)), ((), ())),
                        preferred_element_type=jnp.float32)
    h_ref[...] = h
    s = lax.dot_general(h, m_ref[...], (((1,), (0,)), ((), ())),
                        preferred_element_type=jnp.float32) + b_ref[...]
    s_ref[...] = s
    mx_ref[...] = jnp.max(s, axis=0, keepdims=True)


def _tc_mid(o, dex, W, M, b):
    k = M.shape[1]
    return pl.pallas_call(
        _tc_mid_body,
        out_shape=(jax.ShapeDtypeStruct((N, 128), jnp.float32),
                   jax.ShapeDtypeStruct((N, k), jnp.float32),
                   jax.ShapeDtypeStruct((1, k), jnp.float32)),
    )(o, dex, W, M, b)


def _tc_out_body(o_ref, dex_ref, y_ref):
    v = o_ref[...] / (dex_ref[...] + 1e-16)
    m = jnp.max(v, axis=1, keepdims=True)
    lse = jnp.log(jnp.sum(jnp.exp(v - m), axis=1, keepdims=True)) + m
    y_ref[...] = v - lse


def _tc_out(o, dex):
    return pl.pallas_call(
        _tc_out_body,
        out_shape=jax.ShapeDtypeStruct((N, 128), jnp.float32),
    )(o, dex)


def _pad_table(sT):
    return jnp.pad(sT, ((0, 0), (0, NP - N)))


def _leaky(v):
    return jnp.where(v >= 0, v, v * SLOPE)


def kernel(x, edge_index, W1, a1_w, a1_b, W2, a2_w, a2_b):
    row = edge_index[0].astype(jnp.int32)
    col = edge_index[1].astype(jnp.int32)
    col = jnp.where(row != col, col, jnp.int32(N))
    loop = jnp.arange(N, dtype=jnp.int32)
    pad_n = EPAD - EP
    src = jnp.concatenate([row, loop, jnp.zeros((pad_n,), jnp.int32)])
    dst = jnp.concatenate([col, loop, jnp.full((pad_n,), N, jnp.int32)])
    edges = jnp.stack([dst.reshape(-1, C), src.reshape(-1, C)],
                      axis=1).reshape(-1)

    H1, F1 = 4, 32
    eye1 = jnp.eye(H1, dtype=jnp.float32)
    M1 = jnp.concatenate([jnp.kron(eye1, a1_w[0, :F1][:, None]),
                          jnp.kron(eye1, a1_w[0, F1:][:, None])], axis=1)
    b1 = jnp.concatenate([jnp.broadcast_to(a1_b, (H1,)),
                          jnp.zeros((H1,), jnp.float32)])[None, :]
    h1, s1, mx1 = _tc_in(x, W1, M1, b1)

    sdst1 = _pad_table(s1[:, :H1].T)
    ssrc1 = _pad_table(s1[:, H1:].T)
    ce1 = _leaky(mx1[0, :H1] + mx1[0, H1:])
    carr1 = jnp.broadcast_to(ce1[:, None], (H1, 16))
    hsl1 = h1.T
    acc1, den1 = _sc_gat4(edges, sdst1, ssrc1, hsl1, carr1)
    o1 = acc1[:, :N].T
    dex1 = jnp.broadcast_to(den1[:, :N].T[:, :, None],
                            (N, H1, 128 // H1)).reshape(N, 128)

    H2, F2 = 1, 128
    M2 = jnp.stack([a2_w[0, :F2], a2_w[0, F2:]], axis=1)
    b2 = jnp.stack([a2_b[0], jnp.float32(0)])[None, :]
    h2, s2, mx2 = _tc_mid(o1, dex1, W2, M2, b2)

    sdst2 = _pad_table(s2[:, :1].T)
    ssrc2 = _pad_table(s2[:, 1:].T)
    ce2 = _leaky(mx2[0, :1] + mx2[0, 1:])
    carr2 = jnp.broadcast_to(ce2[:, None], (H2, 16))
    hsl2 = h2.T
    acc2, den2 = _sc_gat1(edges, sdst2, ssrc2, hsl2, carr2)
    o2 = acc2[:, :N].T
    dex2 = jnp.broadcast_to(den2[0, :N][:, None], (N, 128))

    return _tc_out(o2, dex2)


# speedup vs baseline: 21.0653x; 1.1385x over previous
import functools

import jax
import jax.numpy as jnp
from jax import lax
from jax.experimental import pallas as pl
from jax.experimental.pallas import tpu as pltpu
from jax.experimental.pallas import tpu_sc as plsc

N = 10000
E = 320000
EP = E + N
C = 4096
NCH = -2 * (-(EP // C + 1) // 2)
EPAD = NCH * C
NP = 10016
NT = 32
FS = 4
NH4 = N * 4
ND4 = NP * 4
SLOPE = 0.2

_mesh = plsc.VectorSubcoreMesh(
    core_axis_name="c", subcore_axis_name="s", num_cores=2, num_subcores=16)


def _make_sc_gat(H):
    own_stride = NT // H

    @functools.partial(
        pl.kernel,
        mesh=_mesh,
        compiler_params=pltpu.CompilerParams(needs_layout_passes=False),
        out_type=(
            jax.ShapeDtypeStruct((NT * FS, NP), jnp.float32),
            jax.ShapeDtypeStruct((H, NP), jnp.float32),
        ),
        scratch_types=[
            pltpu.VMEM((NP,), jnp.float32),
            pltpu.VMEM((NP,), jnp.float32),
            pltpu.VMEM((N,), jnp.float32),
            pltpu.VMEM((N,), jnp.float32),
            pltpu.VMEM((N,), jnp.float32),
            pltpu.VMEM((N,), jnp.float32),
            pltpu.VMEM((NP,), jnp.float32),
            pltpu.VMEM((NP,), jnp.float32),
            pltpu.VMEM((NP,), jnp.float32),
            pltpu.VMEM((NP,), jnp.float32),
            pltpu.VMEM((NP,), jnp.float32),
            pltpu.VMEM((2 * C,), jnp.int32),
            pltpu.VMEM((2 * C,), jnp.int32),
            pltpu.SemaphoreType.DMA,
            pltpu.SemaphoreType.DMA,
            pltpu.VMEM((16,), jnp.float32),
        ],
    )
    def sc_gat(edge_hbm, sdst_hbm, ssrc_hbm, hsl_hbm, carr_hbm,
               out_hbm, den_hbm,
               sd_tab, ss_tab, h0, h1, h2, h3, o0, o1, o2, o3,
               den_tab, e_buf0, e_buf1, sem0, sem1, c_buf):
        t = lax.axis_index("s") * 2 + lax.axis_index("c")
        head = t // own_stride
        own = (t % own_stride) == 0
        h_tabs = [h0, h1, h2, h3]
        o_tabs = [o0, o1, o2, o3]

        pltpu.sync_copy(sdst_hbm.at[head], sd_tab)
        pltpu.sync_copy(ssrc_hbm.at[head], ss_tab)
        for f in range(FS):
            pltpu.sync_copy(hsl_hbm.at[t * FS + f], h_tabs[f])
        pltpu.sync_copy(carr_hbm.at[head], c_buf)
        cv = c_buf[...]
        ownm = jnp.broadcast_to((t % own_stride).astype(jnp.int32), (16,)) == 0
        zf = jnp.zeros((16,), jnp.float32)

        for fz in range(FS):
            @plsc.parallel_loop(0, NP // 16, unroll=4)
            def zero_out(i, fz=fz):
                o_tabs[fz][pl.ds(i * 16, 16)] = zf

        @plsc.parallel_loop(0, NP // 16, unroll=4)
        def zero_den(i):
            den_tab[pl.ds(i * 16, 16)] = zf

        bufs = ((e_buf0, sem0), (e_buf1, sem1))
        pltpu.async_copy(edge_hbm.at[pl.ds(0, 2 * C)], e_buf0, sem0)
        pltpu.async_copy(edge_hbm.at[pl.ds(2 * C, 2 * C)], e_buf1, sem1)

        def chunk2(g, carry):
            b = g * 2
            for k in range(2):
                e_buf, sem = bufs[k]
                pltpu.make_async_copy(
                    edge_hbm.at[pl.ds(0, 2 * C)], e_buf, sem).wait()

                @plsc.parallel_loop(0, C // 16, unroll=8)
                def edge_block(j):
                    dv = e_buf[pl.ds(j * 16, 16)]
                    sv = e_buf[pl.ds(C + j * 16, 16)]
                    sd = plsc.load_gather(sd_tab, [dv])
                    ss = plsc.load_gather(ss_tab, [sv])
                    z = sd + ss
                    e = jnp.where(z >= 0, z, z * SLOPE) - cv
                    p = jnp.exp(e)
                    for f in range(FS):
                        hf = plsc.load_gather(h_tabs[f], [sv])
                        plsc.addupdate_scatter(o_tabs[f], [dv], p * hf)
                    plsc.addupdate_scatter(den_tab, [dv], p, mask=ownm)

                pltpu.async_copy(
                    edge_hbm.at[pl.ds((b + k + 2) * 2 * C, 2 * C)], e_buf, sem)
            return carry
        lax.fori_loop(0, NCH // 2, chunk2, 0)
        pltpu.make_async_copy(edge_hbm.at[pl.ds(0, 2 * C)], e_buf0, sem0).wait()
        pltpu.make_async_copy(edge_hbm.at[pl.ds(0, 2 * C)], e_buf1, sem1).wait()

        for f in range(FS):
            pltpu.sync_copy(o_tabs[f], out_hbm.at[t * FS + f])

        @pl.when(own)
        def _():
            pltpu.sync_copy(den_tab, den_hbm.at[head])

    return sc_gat


_sc_gat4 = _make_sc_gat(4)
_sc_gat1 = _make_sc_gat(1)


def _tc_in_body(x_ref, w_ref, m_ref, b_ref, h_ref, s_ref, mx_ref):
    h = lax.dot_general(x_ref[...], w_ref[...], (((1,), (1,)), ((), ())),
                        preferred_element_type=jnp.float32)
    h_ref[...] = h
    s = lax.dot_general(h, m_ref[...], (((1,), (0,)), ((), ())),
                        preferred_element_type=jnp.float32) + b_ref[...]
    s_ref[...] = s
    mx_ref[...] = jnp.max(s, axis=0, keepdims=True)


def _tc_in(x, W, M, b):
    k = M.shape[1]
    return pl.pallas_call(
        _tc_in_body,
        out_shape=(jax.ShapeDtypeStruct((N, 128), jnp.float32),
                   jax.ShapeDtypeStruct((N, k), jnp.float32),
                   jax.ShapeDtypeStruct((1, k), jnp.float32)),
    )(x, W, M, b)


def _tc_mid_body(o_ref, dex_ref, w_ref, m_ref, b_ref, h_ref, s_ref, mx_ref):
    xin = o_ref[...] / (dex_ref[...] + 1e-16)
    act = jnp.where(xin > 0, xin, jnp.exp(xin) - 1.0)
    h = lax.dot_general(act, w_ref[...], (((1,), (1,)), ((), ())),
                        preferred_element_type=jnp.float32)
    h_ref[...] = h
    s = lax.dot_general(h, m_ref[...], (((1,), (0,)), ((), ())),
                        preferred_element_type=jnp.float32) + b_ref[...]
    s_ref[...] = s
    mx_ref[...] = jnp.max(s, axis=0, keepdims=True)


def _tc_mid(o, dex, W, M, b):
    k = M.shape[1]
    return pl.pallas_call(
        _tc_mid_body,
        out_shape=(jax.ShapeDtypeStruct((N, 128), jnp.float32),
                   jax.ShapeDtypeStruct((N, k), jnp.float32),
                   jax.ShapeDtypeStruct((1, k), jnp.float32)),
    )(o, dex, W, M, b)


def _tc_out_body(o_ref, dex_ref, y_ref):
    v = o_ref[...] / (dex_ref[...] + 1e-16)
    m = jnp.max(v, axis=1, keepdims=True)
    lse = jnp.log(jnp.sum(jnp.exp(v - m), axis=1, keepdims=True)) + m
    y_ref[...] = v - lse


def _tc_out(o, dex):
    return pl.pallas_call(
        _tc_out_body,
        out_shape=jax.ShapeDtypeStruct((N, 128), jnp.float32),
    )(o, dex)


def _pad_table(sT):
    return jnp.pad(sT, ((0, 0), (0, NP - N)))


def _leaky(v):
    return jnp.where(v >= 0, v, v * SLOPE)


def kernel(x, edge_index, W1, a1_w, a1_b, W2, a2_w, a2_b):
    row = edge_index[0].astype(jnp.int32)
    col = edge_index[1].astype(jnp.int32)
    col = jnp.where(row != col, col, jnp.int32(N))
    loop = jnp.arange(N, dtype=jnp.int32)
    pad_n = EPAD - EP
    src = jnp.concatenate([row, loop, jnp.zeros((pad_n,), jnp.int32)])
    dst = jnp.concatenate([col, loop, jnp.full((pad_n,), N, jnp.int32)])
    edges = jnp.stack([dst.reshape(-1, C), src.reshape(-1, C)],
                      axis=1).reshape(-1)
    edges = jnp.concatenate([edges, jnp.zeros((4 * C,), jnp.int32)])

    H1, F1 = 4, 32
    eye1 = jnp.eye(H1, dtype=jnp.float32)
    M1 = jnp.concatenate([jnp.kron(eye1, a1_w[0, :F1][:, None]),
                          jnp.kron(eye1, a1_w[0, F1:][:, None])], axis=1)
    b1 = jnp.concatenate([jnp.broadcast_to(a1_b, (H1,)),
                          jnp.zeros((H1,), jnp.float32)])[None, :]
    h1, s1, mx1 = _tc_in(x, W1, M1, b1)

    sdst1 = _pad_table(s1[:, :H1].T)
    ssrc1 = _pad_table(s1[:, H1:].T)
    ce1 = _leaky(mx1[0, :H1] + mx1[0, H1:])
    carr1 = jnp.broadcast_to(ce1[:, None], (H1, 16))
    hsl1 = h1.T
    acc1, den1 = _sc_gat4(edges, sdst1, ssrc1, hsl1, carr1)
    o1 = acc1[:, :N].T
    dex1 = jnp.broadcast_to(den1[:, :N].T[:, :, None],
                            (N, H1, 128 // H1)).reshape(N, 128)

    H2, F2 = 1, 128
    M2 = jnp.stack([a2_w[0, :F2], a2_w[0, F2:]], axis=1)
    b2 = jnp.stack([a2_b[0], jnp.float32(0)])[None, :]
    h2, s2, mx2 = _tc_mid(o1, dex1, W2, M2, b2)

    sdst2 = _pad_table(s2[:, :1].T)
    ssrc2 = _pad_table(s2[:, 1:].T)
    ce2 = _leaky(mx2[0, :1] + mx2[0, 1:])
    carr2 = jnp.broadcast_to(ce2[:, None], (H2, 16))
    hsl2 = h2.T
    acc2, den2 = _sc_gat1(edges, sdst2, ssrc2, hsl2, carr2)
    o2 = acc2[:, :N].T
    dex2 = jnp.broadcast_to(den2[0, :N][:, None], (N, 128))

    return _tc_out(o2, dex2)


# speedup vs baseline: 21.1607x; 1.0045x over previous
import functools

import jax
import jax.numpy as jnp
from jax import lax
from jax.experimental import pallas as pl
from jax.experimental.pallas import tpu as pltpu
from jax.experimental.pallas import tpu_sc as plsc

N = 10000
E = 320000
EP = E + N
C = 4096
NCH = -2 * (-(EP // C + 1) // 2)
EPAD = NCH * C
NP = 10016
NT = 32
FS = 4
NH4 = N * 4
ND4 = NP * 4
SLOPE = 0.2

_mesh = plsc.VectorSubcoreMesh(
    core_axis_name="c", subcore_axis_name="s", num_cores=2, num_subcores=16)


def _make_sc_gat(H):
    own_stride = NT // H

    @functools.partial(
        pl.kernel,
        mesh=_mesh,
        compiler_params=pltpu.CompilerParams(needs_layout_passes=False),
        out_type=(
            jax.ShapeDtypeStruct((NT * FS, NP), jnp.float32),
            jax.ShapeDtypeStruct((H, NP), jnp.float32),
        ),
        scratch_types=[
            pltpu.VMEM((NP,), jnp.float32),
            pltpu.VMEM((NP,), jnp.float32),
            pltpu.VMEM((N,), jnp.float32),
            pltpu.VMEM((N,), jnp.float32),
            pltpu.VMEM((N,), jnp.float32),
            pltpu.VMEM((N,), jnp.float32),
            pltpu.VMEM((NP,), jnp.float32),
            pltpu.VMEM((NP,), jnp.float32),
            pltpu.VMEM((NP,), jnp.float32),
            pltpu.VMEM((NP,), jnp.float32),
            pltpu.VMEM((NP,), jnp.float32),
            pltpu.VMEM((2 * C,), jnp.int32),
            pltpu.VMEM((2 * C,), jnp.int32),
            pltpu.SemaphoreType.DMA,
            pltpu.SemaphoreType.DMA,
            pltpu.VMEM((16,), jnp.float32),
        ],
    )
    def sc_gat(edge_hbm, sdst_hbm, ssrc_hbm, hsl_hbm, carr_hbm,
               out_hbm, den_hbm,
               sd_tab, ss_tab, h0, h1, h2, h3, o0, o1, o2, o3,
               den_tab, e_buf0, e_buf1, sem0, sem1, c_buf):
        t = lax.axis_index("s") * 2 + lax.axis_index("c")
        head = t // own_stride
        own = (t % own_stride) == 0
        h_tabs = [h0, h1, h2, h3]
        o_tabs = [o0, o1, o2, o3]

        pltpu.sync_copy(sdst_hbm.at[head], sd_tab)
        pltpu.sync_copy(ssrc_hbm.at[head], ss_tab)
        for f in range(FS):
            pltpu.sync_copy(hsl_hbm.at[t * FS + f], h_tabs[f])
        pltpu.sync_copy(carr_hbm.at[head], c_buf)
        cv = c_buf[...]
        ownm = jnp.broadcast_to((t % own_stride).astype(jnp.int32), (16,)) == 0
        zf = jnp.zeros((16,), jnp.float32)

        for fz in range(FS):
            @plsc.parallel_loop(0, NP // 16, unroll=4)
            def zero_out(i, fz=fz):
                o_tabs[fz][pl.ds(i * 16, 16)] = zf

        @plsc.parallel_loop(0, NP // 16, unroll=4)
        def zero_den(i):
            den_tab[pl.ds(i * 16, 16)] = zf

        bufs = ((e_buf0, sem0), (e_buf1, sem1))
        pltpu.async_copy(edge_hbm.at[pl.ds(0, 2 * C)], e_buf0, sem0)
        pltpu.async_copy(edge_hbm.at[pl.ds(2 * C, 2 * C)], e_buf1, sem1)

        def chunk2(g, carry):
            b = g * 2
            for k in range(2):
                e_buf, sem = bufs[k]
                pltpu.make_async_copy(
                    edge_hbm.at[pl.ds(0, 2 * C)], e_buf, sem).wait()

                @plsc.parallel_loop(0, C // 16, unroll=16)
                def edge_block(j):
                    dv = e_buf[pl.ds(j * 16, 16)]
                    sv = e_buf[pl.ds(C + j * 16, 16)]
                    sd = plsc.load_gather(sd_tab, [dv])
                    ss = plsc.load_gather(ss_tab, [sv])
                    z = sd + ss
                    e = jnp.where(z >= 0, z, z * SLOPE) - cv
                    p = jnp.exp(e)
                    for f in range(FS):
                        hf = plsc.load_gather(h_tabs[f], [sv])
                        plsc.addupdate_scatter(o_tabs[f], [dv], p * hf)
                    plsc.addupdate_scatter(den_tab, [dv], p, mask=ownm)

                pltpu.async_copy(
                    edge_hbm.at[pl.ds((b + k + 2) * 2 * C, 2 * C)], e_buf, sem)
            return carry
        lax.fori_loop(0, NCH // 2, chunk2, 0)
        pltpu.make_async_copy(edge_hbm.at[pl.ds(0, 2 * C)], e_buf0, sem0).wait()
        pltpu.make_async_copy(edge_hbm.at[pl.ds(0, 2 * C)], e_buf1, sem1).wait()

        for f in range(FS):
            pltpu.sync_copy(o_tabs[f], out_hbm.at[t * FS + f])

        @pl.when(own)
        def _():
            pltpu.sync_copy(den_tab, den_hbm.at[head])

    return sc_gat


_sc_gat4 = _make_sc_gat(4)
_sc_gat1 = _make_sc_gat(1)


def _tc_in_body(x_ref, w_ref, m_ref, b_ref, h_ref, s_ref, mx_ref):
    h = lax.dot_general(x_ref[...], w_ref[...], (((1,), (1,)), ((), ())),
                        preferred_element_type=jnp.float32)
    h_ref[...] = h
    s = lax.dot_general(h, m_ref[...], (((1,), (0,)), ((), ())),
                        preferred_element_type=jnp.float32) + b_ref[...]
    s_ref[...] = s
    mx_ref[...] = jnp.max(s, axis=0, keepdims=True)


def _tc_in(x, W, M, b):
    k = M.shape[1]
    return pl.pallas_call(
        _tc_in_body,
        out_shape=(jax.ShapeDtypeStruct((N, 128), jnp.float32),
                   jax.ShapeDtypeStruct((N, k), jnp.float32),
                   jax.ShapeDtypeStruct((1, k), jnp.float32)),
    )(x, W, M, b)


def _tc_mid_body(o_ref, dex_ref, w_ref, m_ref, b_ref, h_ref, s_ref, mx_ref):
    xin = o_ref[...] / (dex_ref[...] + 1e-16)
    act = jnp.where(xin > 0, xin, jnp.exp(xin) - 1.0)
    h = lax.dot_general(act, w_ref[...], (((1,), (1,)), ((), ())),
                        preferred_element_type=jnp.float32)
    h_ref[...] = h
    s = lax.dot_general(h, m_ref[...], (((1,), (0,)), ((), ())),
                        preferred_element_type=jnp.float32) + b_ref[...]
    s_ref[...] = s
    mx_ref[...] = jnp.max(s, axis=0, keepdims=True)


def _tc_mid(o, dex, W, M, b):
    k = M.shape[1]
    return pl.pallas_call(
        _tc_mid_body,
        out_shape=(jax.ShapeDtypeStruct((N, 128), jnp.float32),
                   jax.ShapeDtypeStruct((N, k), jnp.float32),
                   jax.ShapeDtypeStruct((1, k), jnp.float32)),
    )(o, dex, W, M, b)


def _tc_out_body(o_ref, dex_ref, y_ref):
    v = o_ref[...] / (dex_ref[...] + 1e-16)
    m = jnp.max(v, axis=1, keepdims=True)
    lse = jnp.log(jnp.sum(jnp.exp(v - m), axis=1, keepdims=True)) + m
    y_ref[...] = v - lse


def _tc_out(o, dex):
    return pl.pallas_call(
        _tc_out_body,
        out_shape=jax.ShapeDtypeStruct((N, 128), jnp.float32),
    )(o, dex)


def _pad_table(sT):
    return jnp.pad(sT, ((0, 0), (0, NP - N)))


def _leaky(v):
    return jnp.where(v >= 0, v, v * SLOPE)


def kernel(x, edge_index, W1, a1_w, a1_b, W2, a2_w, a2_b):
    row = edge_index[0].astype(jnp.int32)
    col = edge_index[1].astype(jnp.int32)
    col = jnp.where(row != col, col, jnp.int32(N))
    loop = jnp.arange(N, dtype=jnp.int32)
    pad_n = EPAD - EP
    src = jnp.concatenate([row, loop, jnp.zeros((pad_n,), jnp.int32)])
    dst = jnp.concatenate([col, loop, jnp.full((pad_n,), N, jnp.int32)])
    edges = jnp.stack([dst.reshape(-1, C), src.reshape(-1, C)],
                      axis=1).reshape(-1)
    edges = jnp.concatenate([edges, jnp.zeros((4 * C,), jnp.int32)])

    H1, F1 = 4, 32
    eye1 = jnp.eye(H1, dtype=jnp.float32)
    M1 = jnp.concatenate([jnp.kron(eye1, a1_w[0, :F1][:, None]),
                          jnp.kron(eye1, a1_w[0, F1:][:, None])], axis=1)
    b1 = jnp.concatenate([jnp.broadcast_to(a1_b, (H1,)),
                          jnp.zeros((H1,), jnp.float32)])[None, :]
    h1, s1, mx1 = _tc_in(x, W1, M1, b1)

    sdst1 = _pad_table(s1[:, :H1].T)
    ssrc1 = _pad_table(s1[:, H1:].T)
    ce1 = _leaky(mx1[0, :H1] + mx1[0, H1:])
    carr1 = jnp.broadcast_to(ce1[:, None], (H1, 16))
    hsl1 = h1.T
    acc1, den1 = _sc_gat4(edges, sdst1, ssrc1, hsl1, carr1)
    o1 = acc1[:, :N].T
    dex1 = jnp.broadcast_to(den1[:, :N].T[:, :, None],
                            (N, H1, 128 // H1)).reshape(N, 128)

    H2, F2 = 1, 128
    M2 = jnp.stack([a2_w[0, :F2], a2_w[0, F2:]], axis=1)
    b2 = jnp.stack([a2_b[0], jnp.float32(0)])[None, :]
    h2, s2, mx2 = _tc_mid(o1, dex1, W2, M2, b2)

    sdst2 = _pad_table(s2[:, :1].T)
    ssrc2 = _pad_table(s2[:, 1:].T)
    ce2 = _leaky(mx2[0, :1] + mx2[0, 1:])
    carr2 = jnp.broadcast_to(ce2[:, None], (H2, 16))
    hsl2 = h2.T
    acc2, den2 = _sc_gat1(edges, sdst2, ssrc2, hsl2, carr2)
    o2 = acc2[:, :N].T
    dex2 = jnp.broadcast_to(den2[0, :N][:, None], (N, 128))

    return _tc_out(o2, dex2)
